# Initial kernel scaffold; baseline (speedup 1.0000x reference)
import jax
import jax.numpy as jnp
from jax.experimental import pallas as pl


def kernel(x, edge_index, W1, b1, W2, b2):
    raise NotImplementedError("write your pallas kernel here")


# speedup vs baseline: 8.5950x; 8.5950x over previous
import functools

import jax
import jax.numpy as jnp
from jax import lax
from jax.experimental import pallas as pl
from jax.experimental.pallas import tpu as pltpu
from jax.experimental.pallas import tpu_sc as plsc

NC = 2
NS = 16
LB = 128


def _mesh():
    return plsc.VectorSubcoreMesh(core_axis_name="c", subcore_axis_name="s")


_SC_PARAMS = pltpu.CompilerParams(use_tc_tiling_on_sc=False)


def _make_deg_kernel(Np, nb, rows_pt):
    @functools.partial(
        pl.kernel,
        out_type=jax.ShapeDtypeStruct((NC, Np), jnp.float32),
        mesh=_mesh(),
        scratch_types=[
            pltpu.VMEM_SHARED((Np,), jnp.float32),
            pltpu.VMEM((nb, LB), jnp.int32),
            pltpu.VMEM((LB,), jnp.float32),
        ],
        compiler_params=_SC_PARAMS,
    )
    def deg_kernel(dst_hbm, zeros_hbm, ones_hbm, out_hbm, acc_sh, idx_v, ones_v):
        c = lax.axis_index("c")
        s = lax.axis_index("s")
        pltpu.sync_copy(dst_hbm.at[c, s], idx_v)
        pltpu.sync_copy(ones_hbm, ones_v)
        pltpu.sync_copy(zeros_hbm, acc_sh.at[pl.ds(s * rows_pt, rows_pt)])
        plsc.subcore_barrier()

        def body(j, carry):
            pltpu.sync_copy(ones_v, acc_sh.at[idx_v.at[j]], add=True)
            return carry

        lax.fori_loop(0, nb, body, 0)
        plsc.subcore_barrier()
        pltpu.sync_copy(acc_sh.at[pl.ds(s * rows_pt, rows_pt)],
                        out_hbm.at[c, pl.ds(s * rows_pt, rows_pt)])

    return deg_kernel


def _make_agg_kernel(Np, Fc, nb, rows_pt):
    @functools.partial(
        pl.kernel,
        out_type=jax.ShapeDtypeStruct((NC, Np, Fc), jnp.float32),
        mesh=_mesh(),
        scratch_types=[
            pltpu.VMEM_SHARED((Np, Fc), jnp.float32),
            pltpu.VMEM((nb, LB), jnp.int32),
            pltpu.VMEM((nb, LB), jnp.int32),
            pltpu.VMEM((LB, Fc), jnp.float32),
            pltpu.SemaphoreType.DMA,
        ],
        compiler_params=_SC_PARAMS,
    )
    def agg_kernel(gflat_hbm, src_hbm, dst_hbm, out_hbm,
                   acc_sh, src_v, dst_v, buf, sem):
        c = lax.axis_index("c")
        s = lax.axis_index("s")
        pltpu.sync_copy(src_hbm.at[c, s], src_v)
        pltpu.sync_copy(dst_hbm.at[s], dst_v)
        base = c * Np + s * rows_pt
        pltpu.sync_copy(gflat_hbm.at[pl.ds(base, rows_pt)],
                        acc_sh.at[pl.ds(s * rows_pt, rows_pt)])
        plsc.subcore_barrier()

        def body(j, carry):
            pltpu.async_copy(gflat_hbm.at[src_v.at[j]], buf, sem).wait()
            pltpu.sync_copy(buf, acc_sh.at[dst_v.at[j]], add=True)
            return carry

        lax.fori_loop(0, nb, body, 0)
        plsc.subcore_barrier()
        pltpu.sync_copy(acc_sh.at[pl.ds(s * rows_pt, rows_pt)],
                        out_hbm.at[c, pl.ds(s * rows_pt, rows_pt)])

    return agg_kernel


def kernel(x, edge_index, W1, b1, W2, b2):
    N, Fin = x.shape
    E = edge_index.shape[1]
    H = W1.shape[1]
    C = W2.shape[1]
    Fc1 = H // 2
    Fc2 = C // 2

    Np = -(-N // (NS * LB)) * (NS * LB)
    rows_pt = Np // NS
    Ep = -(-E // (NC * NS * LB)) * (NC * NS * LB)
    nb_agg = Ep // (NS * LB)
    nb_deg = Ep // (NC * NS * LB)
    Bn = Np // 8

    pad = Ep - E
    src = edge_index[0]
    dst = edge_index[1]
    srcp = jnp.concatenate([src, jnp.full((pad,), N, jnp.int32)])
    dstp = jnp.concatenate([dst, jnp.full((pad,), N, jnp.int32)])
    src2 = jnp.stack([srcp, srcp + Np]).reshape(NC, NS, nb_agg, LB)
    dst_t = dstp.reshape(NS, nb_agg, LB)
    dst_deg = dstp.reshape(NC, NS, nb_deg, LB)
    zeros_rows = jnp.zeros((rows_pt,), jnp.float32)
    ones_lb = jnp.ones((LB,), jnp.float32)
    x_pad = jnp.pad(x, ((0, Np - N), (0, 0)))
    b1r = b1.reshape(1, H)
    b2r = b2.reshape(1, C)

    deg2 = _make_deg_kernel(Np, nb_deg, rows_pt)(dst_deg, zeros_rows, ones_lb)

    def l1_body(deg_ref, x_ref, w_ref, g_ref):
        dinv = lax.rsqrt(deg_ref[0, :] + deg_ref[1, :] + 1.0)
        h = jnp.dot(x_ref[...], w_ref[...], preferred_element_type=jnp.float32)
        g = h * dinv[:, None]
        g_ref[0] = g[:, :Fc1]
        g_ref[1] = g[:, Fc1:]

    g1 = pl.pallas_call(
        l1_body,
        grid=(Np // Bn,),
        in_specs=[
            pl.BlockSpec((NC, Bn), lambda i: (0, i)),
            pl.BlockSpec((Bn, Fin), lambda i: (i, 0)),
            pl.BlockSpec((Fin, H), lambda i: (0, 0)),
        ],
        out_specs=pl.BlockSpec((NC, Bn, Fc1), lambda i: (0, i, 0)),
        out_shape=jax.ShapeDtypeStruct((NC, Np, Fc1), jnp.float32),
    )(deg2, x_pad, W1)

    agg1 = _make_agg_kernel(Np, Fc1, nb_agg, rows_pt)(
        g1.reshape(NC * Np, Fc1), src2, dst_t)

    def l2_body(deg_ref, a_ref, b_ref, w_ref, g_ref):
        i = pl.program_id(0)
        dinv = lax.rsqrt(deg_ref[0, :] + deg_ref[1, :] + 1.0)
        agg = jnp.concatenate([a_ref[0], a_ref[1]], axis=1)
        t = jnp.maximum(agg * dinv[:, None] + b_ref[...], 0.0)
        row = i * Bn + lax.broadcasted_iota(jnp.int32, (Bn, 1), 0)
        t = jnp.where(row < N, t, 0.0)
        g = jnp.dot(t, w_ref[...], preferred_element_type=jnp.float32)
        g = g * dinv[:, None]
        g_ref[0] = g[:, :Fc2]
        g_ref[1] = g[:, Fc2:]

    g2 = pl.pallas_call(
        l2_body,
        grid=(Np // Bn,),
        in_specs=[
            pl.BlockSpec((NC, Bn), lambda i: (0, i)),
            pl.BlockSpec((NC, Bn, Fc1), lambda i: (0, i, 0)),
            pl.BlockSpec((1, H), lambda i: (0, 0)),
            pl.BlockSpec((H, C), lambda i: (0, 0)),
        ],
        out_specs=pl.BlockSpec((NC, Bn, Fc2), lambda i: (0, i, 0)),
        out_shape=jax.ShapeDtypeStruct((NC, Np, Fc2), jnp.float32),
    )(deg2, agg1, b1r, W2)

    agg2 = _make_agg_kernel(Np, Fc2, nb_agg, rows_pt)(
        g2.reshape(NC * Np, Fc2), src2, dst_t)

    def l3_body(deg_ref, a_ref, b_ref, o_ref):
        dinv = lax.rsqrt(deg_ref[0, :] + deg_ref[1, :] + 1.0)
        agg = jnp.concatenate([a_ref[0], a_ref[1]], axis=1)
        o_ref[...] = agg * dinv[:, None] + b_ref[...]

    out = pl.pallas_call(
        l3_body,
        grid=(Np // Bn,),
        in_specs=[
            pl.BlockSpec((NC, Bn), lambda i: (0, i)),
            pl.BlockSpec((NC, Bn, Fc2), lambda i: (0, i, 0)),
            pl.BlockSpec((1, C), lambda i: (0, 0)),
        ],
        out_specs=pl.BlockSpec((Bn, C), lambda i: (i, 0)),
        out_shape=jax.ShapeDtypeStruct((Np, C), jnp.float32),
    )(deg2, agg2, b2r)

    return out[:N]


# speedup vs baseline: 8.6147x; 1.0023x over previous
import functools

import jax
import jax.numpy as jnp
from jax import lax
from jax.experimental import pallas as pl
from jax.experimental.pallas import tpu as pltpu
from jax.experimental.pallas import tpu_sc as plsc

NC = 2
NS = 16
LB = 128


def _mesh():
    return plsc.VectorSubcoreMesh(core_axis_name="c", subcore_axis_name="s")


_SC_PARAMS = pltpu.CompilerParams(use_tc_tiling_on_sc=False)


def _make_deg_kernel(Np, nb, rows_pt):
    @functools.partial(
        pl.kernel,
        out_type=jax.ShapeDtypeStruct((NC, Np), jnp.float32),
        mesh=_mesh(),
        scratch_types=[
            pltpu.VMEM_SHARED((Np,), jnp.float32),
            pltpu.VMEM((nb, LB), jnp.int32),
            pltpu.VMEM((LB,), jnp.float32),
        ],
        compiler_params=_SC_PARAMS,
    )
    def deg_kernel(dst_hbm, zeros_hbm, ones_hbm, out_hbm, acc_sh, idx_v, ones_v):
        c = lax.axis_index("c")
        s = lax.axis_index("s")
        pltpu.sync_copy(dst_hbm.at[c, s], idx_v)
        pltpu.sync_copy(ones_hbm, ones_v)
        pltpu.sync_copy(zeros_hbm, acc_sh.at[pl.ds(s * rows_pt, rows_pt)])
        plsc.subcore_barrier()

        def body(j, carry):
            pltpu.sync_copy(ones_v, acc_sh.at[idx_v.at[j]], add=True)
            return carry

        lax.fori_loop(0, nb, body, 0)
        plsc.subcore_barrier()
        pltpu.sync_copy(acc_sh.at[pl.ds(s * rows_pt, rows_pt)],
                        out_hbm.at[c, pl.ds(s * rows_pt, rows_pt)])

    return deg_kernel


def _make_agg_kernel(Np, Fc, nb, rows_pt):
    nbuf = 2 if Fc >= 128 else 8
    assert nb % nbuf == 0

    @functools.partial(
        pl.kernel,
        out_type=jax.ShapeDtypeStruct((NC, Np, Fc), jnp.float32),
        mesh=_mesh(),
        scratch_types=[
            pltpu.VMEM_SHARED((Np, Fc), jnp.float32),
            pltpu.VMEM((nb, LB), jnp.int32),
            pltpu.VMEM((nbuf, LB), jnp.int32),
            pltpu.VMEM((nbuf, LB), jnp.int32),
            pltpu.VMEM((nbuf, LB, Fc), jnp.float32),
            pltpu.SemaphoreType.DMA((nbuf,)),
            pltpu.SemaphoreType.DMA((nbuf,)),
        ],
        compiler_params=_SC_PARAMS,
    )
    def agg_kernel(gflat_hbm, packed_hbm, out_hbm,
                   acc_sh, pk_v, src_v, dst_v, buf, gsem, ssem):
        c = lax.axis_index("c")
        s = lax.axis_index("s")
        off = c * Np
        pltpu.sync_copy(packed_hbm.at[s], pk_v)
        base = off + s * rows_pt
        pltpu.sync_copy(gflat_hbm.at[pl.ds(base, rows_pt)],
                        acc_sh.at[pl.ds(s * rows_pt, rows_pt)])
        plsc.subcore_barrier()

        def unpack(j, b):
            for k in range(LB // 16):
                p = pk_v[j, pl.ds(16 * k, 16)]
                src_v[b, pl.ds(16 * k, 16)] = (p & 0x7FFF) + off
                dst_v[b, pl.ds(16 * k, 16)] = lax.shift_right_logical(p, 15)

        for b in range(nbuf):
            unpack(b, b)
            pltpu.async_copy(gflat_hbm.at[src_v.at[b]], buf.at[b], gsem.at[b])

        def body(it, carry):
            j0 = it * nbuf
            for b in range(nbuf):
                j = j0 + b
                pltpu.make_async_copy(
                    gflat_hbm.at[src_v.at[b]], buf.at[b], gsem.at[b]).wait()
                pltpu.async_copy(
                    buf.at[b], acc_sh.at[dst_v.at[b]], ssem.at[b], add=True)
            for b in range(nbuf):
                jn = j0 + nbuf + b

                @pl.when(jn < nb)
                def _():
                    pltpu.make_async_copy(
                        buf.at[b], acc_sh.at[dst_v.at[b]], ssem.at[b]).wait()
                    unpack(jn, b)
                    pltpu.async_copy(
                        gflat_hbm.at[src_v.at[b]], buf.at[b], gsem.at[b])

            return carry

        lax.fori_loop(0, nb // nbuf, body, 0)
        for b in range(nbuf):
            pltpu.make_async_copy(
                buf.at[b], acc_sh.at[dst_v.at[b]], ssem.at[b]).wait()
        plsc.subcore_barrier()
        pltpu.sync_copy(acc_sh.at[pl.ds(s * rows_pt, rows_pt)],
                        out_hbm.at[c, pl.ds(s * rows_pt, rows_pt)])

    return agg_kernel


def kernel(x, edge_index, W1, b1, W2, b2):
    N, Fin = x.shape
    E = edge_index.shape[1]
    H = W1.shape[1]
    C = W2.shape[1]
    Fc1 = H // 2
    Fc2 = C // 2

    Np = -(-N // (NS * LB)) * (NS * LB)
    rows_pt = Np // NS
    Ep = -(-E // (NC * NS * LB)) * (NC * NS * LB)
    nb_agg = Ep // (NS * LB)
    nb_deg = Ep // (NC * NS * LB)
    Bn = Np // 8

    pad = Ep - E
    src = edge_index[0]
    dst = edge_index[1]
    srcp = jnp.concatenate([src, jnp.full((pad,), N, jnp.int32)])
    dstp = jnp.concatenate([dst, jnp.full((pad,), N, jnp.int32)])
    packed = (srcp | (dstp << 15)).reshape(NS, nb_agg, LB)
    dst_deg = dstp.reshape(NC, NS, nb_deg, LB)
    zeros_rows = jnp.zeros((rows_pt,), jnp.float32)
    ones_lb = jnp.ones((LB,), jnp.float32)
    x_pad = jnp.pad(x, ((0, Np - N), (0, 0)))
    b1r = b1.reshape(1, H)
    b2r = b2.reshape(1, C)

    deg2 = _make_deg_kernel(Np, nb_deg, rows_pt)(dst_deg, zeros_rows, ones_lb)

    def l1_body(deg_ref, x_ref, w_ref, g_ref):
        dinv = lax.rsqrt(deg_ref[0, :] + deg_ref[1, :] + 1.0)
        h = jnp.dot(x_ref[...], w_ref[...], preferred_element_type=jnp.float32)
        g = h * dinv[:, None]
        g_ref[0] = g[:, :Fc1]
        g_ref[1] = g[:, Fc1:]

    g1 = pl.pallas_call(
        l1_body,
        grid=(Np // Bn,),
        in_specs=[
            pl.BlockSpec((NC, Bn), lambda i: (0, i)),
            pl.BlockSpec((Bn, Fin), lambda i: (i, 0)),
            pl.BlockSpec((Fin, H), lambda i: (0, 0)),
        ],
        out_specs=pl.BlockSpec((NC, Bn, Fc1), lambda i: (0, i, 0)),
        out_shape=jax.ShapeDtypeStruct((NC, Np, Fc1), jnp.float32),
    )(deg2, x_pad, W1)

    agg1 = _make_agg_kernel(Np, Fc1, nb_agg, rows_pt)(
        g1.reshape(NC * Np, Fc1), packed)

    def l2_body(deg_ref, a_ref, b_ref, w_ref, g_ref):
        i = pl.program_id(0)
        dinv = lax.rsqrt(deg_ref[0, :] + deg_ref[1, :] + 1.0)
        agg = jnp.concatenate([a_ref[0], a_ref[1]], axis=1)
        t = jnp.maximum(agg * dinv[:, None] + b_ref[...], 0.0)
        row = i * Bn + lax.broadcasted_iota(jnp.int32, (Bn, 1), 0)
        t = jnp.where(row < N, t, 0.0)
        g = jnp.dot(t, w_ref[...], preferred_element_type=jnp.float32)
        g = g * dinv[:, None]
        g_ref[0] = g[:, :Fc2]
        g_ref[1] = g[:, Fc2:]

    g2 = pl.pallas_call(
        l2_body,
        grid=(Np // Bn,),
        in_specs=[
            pl.BlockSpec((NC, Bn), lambda i: (0, i)),
            pl.BlockSpec((NC, Bn, Fc1), lambda i: (0, i, 0)),
            pl.BlockSpec((1, H), lambda i: (0, 0)),
            pl.BlockSpec((H, C), lambda i: (0, 0)),
        ],
        out_specs=pl.BlockSpec((NC, Bn, Fc2), lambda i: (0, i, 0)),
        out_shape=jax.ShapeDtypeStruct((NC, Np, Fc2), jnp.float32),
    )(deg2, agg1, b1r, W2)

    agg2 = _make_agg_kernel(Np, Fc2, nb_agg, rows_pt)(
        g2.reshape(NC * Np, Fc2), packed)

    def l3_body(deg_ref, a_ref, b_ref, o_ref):
        dinv = lax.rsqrt(deg_ref[0, :] + deg_ref[1, :] + 1.0)
        agg = jnp.concatenate([a_ref[0], a_ref[1]], axis=1)
        o_ref[...] = agg * dinv[:, None] + b_ref[...]

    out = pl.pallas_call(
        l3_body,
        grid=(Np // Bn,),
        in_specs=[
            pl.BlockSpec((NC, Bn), lambda i: (0, i)),
            pl.BlockSpec((NC, Bn, Fc2), lambda i: (0, i, 0)),
            pl.BlockSpec((1, C), lambda i: (0, 0)),
        ],
        out_specs=pl.BlockSpec((Bn, C), lambda i: (i, 0)),
        out_shape=jax.ShapeDtypeStruct((Np, C), jnp.float32),
    )(deg2, agg2, b2r)

    return out[:N]


# speedup vs baseline: 8.8514x; 1.0275x over previous
import functools

import jax
import jax.numpy as jnp
from jax import lax
from jax.experimental import pallas as pl
from jax.experimental.pallas import tpu as pltpu
from jax.experimental.pallas import tpu_sc as plsc

NC = 2
NS = 16
LB = 128


def _mesh():
    return plsc.VectorSubcoreMesh(core_axis_name="c", subcore_axis_name="s")


_SC_PARAMS = pltpu.CompilerParams(use_tc_tiling_on_sc=False)


def _make_deg_kernel(Np, nb, rows_pt):
    @functools.partial(
        pl.kernel,
        out_type=jax.ShapeDtypeStruct((NC, Np), jnp.float32),
        mesh=_mesh(),
        scratch_types=[
            pltpu.VMEM_SHARED((Np,), jnp.float32),
            pltpu.VMEM((nb, LB), jnp.int32),
            pltpu.VMEM((LB,), jnp.float32),
        ],
        compiler_params=_SC_PARAMS,
    )
    def deg_kernel(dst_hbm, zeros_hbm, ones_hbm, out_hbm, acc_sh, idx_v, ones_v):
        c = lax.axis_index("c")
        s = lax.axis_index("s")
        pltpu.sync_copy(dst_hbm.at[c, s], idx_v)
        pltpu.sync_copy(ones_hbm, ones_v)
        pltpu.sync_copy(zeros_hbm, acc_sh.at[pl.ds(s * rows_pt, rows_pt)])
        plsc.subcore_barrier()

        def body(j, carry):
            pltpu.sync_copy(ones_v, acc_sh.at[idx_v.at[j]], add=True)
            return carry

        lax.fori_loop(0, nb, body, 0)
        plsc.subcore_barrier()
        pltpu.sync_copy(acc_sh.at[pl.ds(s * rows_pt, rows_pt)],
                        out_hbm.at[c, pl.ds(s * rows_pt, rows_pt)])

    return deg_kernel


def _make_agg_kernel(Np, Fc, nb, rows_pt, lb, nbuf):
    assert nb % nbuf == 0

    @functools.partial(
        pl.kernel,
        out_type=jax.ShapeDtypeStruct((NC, Np, Fc), jnp.float32),
        mesh=_mesh(),
        scratch_types=[
            pltpu.VMEM_SHARED((Np, Fc), jnp.float32),
            pltpu.VMEM((nb, lb), jnp.int32),
            pltpu.VMEM((nbuf, lb), jnp.int32),
            pltpu.VMEM((nbuf, lb), jnp.int32),
            pltpu.VMEM((nbuf, lb, Fc), jnp.float32),
            pltpu.SemaphoreType.DMA((nbuf,)),
            pltpu.SemaphoreType.DMA((nbuf,)),
        ],
        compiler_params=_SC_PARAMS,
    )
    def agg_kernel(gflat_hbm, packed_hbm, out_hbm,
                   acc_sh, pk_v, src_v, dst_v, buf, gsem, ssem):
        c = lax.axis_index("c")
        s = lax.axis_index("s")
        off = c * Np
        pltpu.sync_copy(packed_hbm.at[s], pk_v)
        base = off + s * rows_pt
        pltpu.sync_copy(gflat_hbm.at[pl.ds(base, rows_pt)],
                        acc_sh.at[pl.ds(s * rows_pt, rows_pt)])
        plsc.subcore_barrier()

        def unpack(j, b):
            for k in range(lb // 16):
                p = pk_v[j, pl.ds(16 * k, 16)]
                src_v[b, pl.ds(16 * k, 16)] = (p & 0x7FFF) + off
                dst_v[b, pl.ds(16 * k, 16)] = lax.shift_right_logical(p, 15)

        for b in range(nbuf):
            unpack(b, b)
            pltpu.async_copy(gflat_hbm.at[src_v.at[b]], buf.at[b], gsem.at[b])

        def body(it, carry):
            j0 = it * nbuf
            for b in range(nbuf):
                j = j0 + b
                pltpu.make_async_copy(
                    gflat_hbm.at[src_v.at[b]], buf.at[b], gsem.at[b]).wait()
                pltpu.async_copy(
                    buf.at[b], acc_sh.at[dst_v.at[b]], ssem.at[b], add=True)
            for b in range(nbuf):
                jn = j0 + nbuf + b

                @pl.when(jn < nb)
                def _():
                    pltpu.make_async_copy(
                        buf.at[b], acc_sh.at[dst_v.at[b]], ssem.at[b]).wait()
                    unpack(jn, b)
                    pltpu.async_copy(
                        gflat_hbm.at[src_v.at[b]], buf.at[b], gsem.at[b])

            return carry

        lax.fori_loop(0, nb // nbuf, body, 0)
        for b in range(nbuf):
            pltpu.make_async_copy(
                buf.at[b], acc_sh.at[dst_v.at[b]], ssem.at[b]).wait()
        plsc.subcore_barrier()
        pltpu.sync_copy(acc_sh.at[pl.ds(s * rows_pt, rows_pt)],
                        out_hbm.at[c, pl.ds(s * rows_pt, rows_pt)])

    return agg_kernel


def kernel(x, edge_index, W1, b1, W2, b2):
    N, Fin = x.shape
    E = edge_index.shape[1]
    H = W1.shape[1]
    C = W2.shape[1]
    Fc1 = H // 2
    Fc2 = C // 2

    Np = -(-N // (NS * LB)) * (NS * LB)
    rows_pt = Np // NS
    Ep = -(-E // (NC * NS * LB)) * (NC * NS * LB)
    nb_agg = Ep // (NS * LB)
    nb_deg = Ep // (NC * NS * LB)
    Bn = Np // 8

    pad = Ep - E
    src = edge_index[0]
    dst = edge_index[1]
    srcp = jnp.concatenate([src, jnp.full((pad,), N, jnp.int32)])
    dstp = jnp.concatenate([dst, jnp.full((pad,), N, jnp.int32)])
    packed = (srcp | (dstp << 15)).reshape(NS, nb_agg, LB)
    dst_deg = dstp.reshape(NC, NS, nb_deg, LB)
    zeros_rows = jnp.zeros((rows_pt,), jnp.float32)
    ones_lb = jnp.ones((LB,), jnp.float32)
    x_pad = jnp.pad(x, ((0, Np - N), (0, 0)))
    b1r = b1.reshape(1, H)
    b2r = b2.reshape(1, C)

    deg2 = _make_deg_kernel(Np, nb_deg, rows_pt)(dst_deg, zeros_rows, ones_lb)

    def l1_body(deg_ref, x_ref, w_ref, g_ref):
        dinv = lax.rsqrt(deg_ref[0, :] + deg_ref[1, :] + 1.0)
        h = jnp.dot(x_ref[...], w_ref[...], preferred_element_type=jnp.float32)
        g = h * dinv[:, None]
        g_ref[0] = g[:, :Fc1]
        g_ref[1] = g[:, Fc1:]

    g1 = pl.pallas_call(
        l1_body,
        grid=(Np // Bn,),
        in_specs=[
            pl.BlockSpec((NC, Bn), lambda i: (0, i)),
            pl.BlockSpec((Bn, Fin), lambda i: (i, 0)),
            pl.BlockSpec((Fin, H), lambda i: (0, 0)),
        ],
        out_specs=pl.BlockSpec((NC, Bn, Fc1), lambda i: (0, i, 0)),
        out_shape=jax.ShapeDtypeStruct((NC, Np, Fc1), jnp.float32),
    )(deg2, x_pad, W1)

    lb1, nbuf1 = 64, 4
    agg1 = _make_agg_kernel(Np, Fc1, Ep // (NS * lb1), rows_pt, lb1, nbuf1)(
        g1.reshape(NC * Np, Fc1), packed.reshape(NS, -1, lb1))

    def l2_body(deg_ref, a_ref, b_ref, w_ref, g_ref):
        i = pl.program_id(0)
        dinv = lax.rsqrt(deg_ref[0, :] + deg_ref[1, :] + 1.0)
        agg = jnp.concatenate([a_ref[0], a_ref[1]], axis=1)
        t = jnp.maximum(agg * dinv[:, None] + b_ref[...], 0.0)
        row = i * Bn + lax.broadcasted_iota(jnp.int32, (Bn, 1), 0)
        t = jnp.where(row < N, t, 0.0)
        g = jnp.dot(t, w_ref[...], preferred_element_type=jnp.float32)
        g = g * dinv[:, None]
        g_ref[0] = g[:, :Fc2]
        g_ref[1] = g[:, Fc2:]

    g2 = pl.pallas_call(
        l2_body,
        grid=(Np // Bn,),
        in_specs=[
            pl.BlockSpec((NC, Bn), lambda i: (0, i)),
            pl.BlockSpec((NC, Bn, Fc1), lambda i: (0, i, 0)),
            pl.BlockSpec((1, H), lambda i: (0, 0)),
            pl.BlockSpec((H, C), lambda i: (0, 0)),
        ],
        out_specs=pl.BlockSpec((NC, Bn, Fc2), lambda i: (0, i, 0)),
        out_shape=jax.ShapeDtypeStruct((NC, Np, Fc2), jnp.float32),
    )(deg2, agg1, b1r, W2)

    lb2, nbuf2 = 64, 8
    agg2 = _make_agg_kernel(Np, Fc2, Ep // (NS * lb2), rows_pt, lb2, nbuf2)(
        g2.reshape(NC * Np, Fc2), packed.reshape(NS, -1, lb2))

    def l3_body(deg_ref, a_ref, b_ref, o_ref):
        dinv = lax.rsqrt(deg_ref[0, :] + deg_ref[1, :] + 1.0)
        agg = jnp.concatenate([a_ref[0], a_ref[1]], axis=1)
        o_ref[...] = agg * dinv[:, None] + b_ref[...]

    out = pl.pallas_call(
        l3_body,
        grid=(Np // Bn,),
        in_specs=[
            pl.BlockSpec((NC, Bn), lambda i: (0, i)),
            pl.BlockSpec((NC, Bn, Fc2), lambda i: (0, i, 0)),
            pl.BlockSpec((1, C), lambda i: (0, 0)),
        ],
        out_specs=pl.BlockSpec((Bn, C), lambda i: (i, 0)),
        out_shape=jax.ShapeDtypeStruct((Np, C), jnp.float32),
    )(deg2, agg2, b2r)

    return out[:N]


# speedup vs baseline: 13.1630x; 1.4871x over previous
import functools

import jax
import jax.numpy as jnp
from jax import lax
from jax.experimental import pallas as pl
from jax.experimental.pallas import tpu as pltpu
from jax.experimental.pallas import tpu_sc as plsc

NC = 2
NS = 16
LB = 128
W = 64


def _mesh():
    return plsc.VectorSubcoreMesh(core_axis_name="c", subcore_axis_name="s")


_SC_PARAMS = pltpu.CompilerParams(use_tc_tiling_on_sc=False)


def _make_deg_kernel(Np, nb, rows_pt):
    @functools.partial(
        pl.kernel,
        out_type=jax.ShapeDtypeStruct((NC, Np), jnp.float32),
        mesh=_mesh(),
        scratch_types=[
            pltpu.VMEM_SHARED((Np,), jnp.float32),
            pltpu.VMEM((nb, LB), jnp.int32),
            pltpu.VMEM((LB,), jnp.float32),
        ],
        compiler_params=_SC_PARAMS,
    )
    def deg_kernel(dst_hbm, zeros_hbm, ones_hbm, out_hbm, acc_sh, idx_v, ones_v):
        c = lax.axis_index("c")
        s = lax.axis_index("s")
        pltpu.sync_copy(dst_hbm.at[c, s], idx_v)
        pltpu.sync_copy(ones_hbm, ones_v)
        pltpu.sync_copy(zeros_hbm, acc_sh.at[pl.ds(s * rows_pt, rows_pt)])
        plsc.subcore_barrier()

        def body(j, carry):
            pltpu.sync_copy(ones_v, acc_sh.at[idx_v.at[j]], add=True)
            return carry

        lax.fori_loop(0, nb, body, 0)
        plsc.subcore_barrier()
        pltpu.sync_copy(acc_sh.at[pl.ds(s * rows_pt, rows_pt)],
                        out_hbm.at[c, pl.ds(s * rows_pt, rows_pt)])

    return deg_kernel


def _make_agg_kernel(Np, npass, nb, rows_pt, nbuf):
    assert nb % nbuf == 0

    @functools.partial(
        pl.kernel,
        out_type=jax.ShapeDtypeStruct((npass * NC, Np, W), jnp.float32),
        mesh=_mesh(),
        scratch_types=[
            pltpu.VMEM_SHARED((Np, W), jnp.float32),
            pltpu.VMEM_SHARED((Np, W), jnp.float32),
            pltpu.VMEM((nb, LB), jnp.int32),
            pltpu.VMEM((nbuf, LB), jnp.int32),
            pltpu.VMEM((nbuf, LB), jnp.int32),
            pltpu.VMEM((nbuf, LB, W), jnp.float32),
            pltpu.SemaphoreType.DMA((nbuf,)),
            pltpu.SemaphoreType.DMA((nbuf,)),
        ],
        compiler_params=_SC_PARAMS,
    )
    def agg_kernel(gch_hbm, packed_hbm, out_hbm,
                   table_sh, acc_sh, pk_v, src_v, dst_v, buf, gsem, ssem):
        c = lax.axis_index("c")
        s = lax.axis_index("s")
        rows = pl.ds(s * rows_pt, rows_pt)
        pltpu.sync_copy(packed_hbm.at[s], pk_v)

        def unpack(j, b):
            for k in range(LB // 16):
                p = pk_v[j, pl.ds(16 * k, 16)]
                src_v[b, pl.ds(16 * k, 16)] = p & 0x7FFF
                dst_v[b, pl.ds(16 * k, 16)] = lax.shift_right_logical(p, 15)

        for p in range(npass):
            ch = p * NC + c
            pltpu.sync_copy(gch_hbm.at[ch, rows], table_sh.at[rows])
            pltpu.sync_copy(gch_hbm.at[ch, rows], acc_sh.at[rows])
            plsc.subcore_barrier()

            for b in range(nbuf):
                unpack(b, b)
                pltpu.async_copy(table_sh.at[src_v.at[b]], buf.at[b],
                                 gsem.at[b])

            def body(it, carry):
                j0 = it * nbuf
                for b in range(nbuf):
                    pltpu.make_async_copy(
                        table_sh.at[src_v.at[b]], buf.at[b],
                        gsem.at[b]).wait()
                    pltpu.async_copy(
                        buf.at[b], acc_sh.at[dst_v.at[b]], ssem.at[b],
                        add=True)
                for b in range(nbuf):
                    jn = j0 + nbuf + b

                    @pl.when(jn < nb)
                    def _():
                        pltpu.make_async_copy(
                            buf.at[b], acc_sh.at[dst_v.at[b]],
                            ssem.at[b]).wait()
                        unpack(jn, b)
                        pltpu.async_copy(
                            table_sh.at[src_v.at[b]], buf.at[b], gsem.at[b])

                return carry

            lax.fori_loop(0, nb // nbuf, body, 0)
            for b in range(nbuf):
                pltpu.make_async_copy(
                    buf.at[b], acc_sh.at[dst_v.at[b]], ssem.at[b]).wait()
            plsc.subcore_barrier()
            pltpu.sync_copy(acc_sh.at[rows], out_hbm.at[ch, rows])
            if p < npass - 1:
                plsc.subcore_barrier()

    return agg_kernel


def kernel(x, edge_index, W1, b1, W2, b2):
    N, Fin = x.shape
    E = edge_index.shape[1]
    H = W1.shape[1]
    C = W2.shape[1]
    np1 = H // (NC * W)
    np2 = C // (NC * W)

    Np = -(-N // (NS * LB)) * (NS * LB)
    rows_pt = Np // NS
    Ep = -(-E // (NC * NS * LB)) * (NC * NS * LB)
    nb_agg = Ep // (NS * LB)
    nb_deg = Ep // (NC * NS * LB)
    Bn = Np // 8

    pad = Ep - E
    src = edge_index[0]
    dst = edge_index[1]
    srcp = jnp.concatenate([src, jnp.full((pad,), N, jnp.int32)])
    dstp = jnp.concatenate([dst, jnp.full((pad,), N, jnp.int32)])
    packed = (srcp | (dstp << 15)).reshape(NS, nb_agg, LB)
    dst_deg = dstp.reshape(NC, NS, nb_deg, LB)
    zeros_rows = jnp.zeros((rows_pt,), jnp.float32)
    ones_lb = jnp.ones((LB,), jnp.float32)
    x_pad = jnp.pad(x, ((0, Np - N), (0, 0)))
    b1r = b1.reshape(1, H)
    b2r = b2.reshape(1, C)
    nch1 = np1 * NC
    nch2 = np2 * NC

    deg2 = _make_deg_kernel(Np, nb_deg, rows_pt)(dst_deg, zeros_rows, ones_lb)

    def l1_body(deg_ref, x_ref, w_ref, g_ref):
        dinv = lax.rsqrt(deg_ref[0, :] + deg_ref[1, :] + 1.0)
        h = jnp.dot(x_ref[...], w_ref[...], preferred_element_type=jnp.float32)
        g = h * dinv[:, None]
        for k in range(nch1):
            g_ref[k] = g[:, W * k:W * (k + 1)]

    g1 = pl.pallas_call(
        l1_body,
        grid=(Np // Bn,),
        in_specs=[
            pl.BlockSpec((NC, Bn), lambda i: (0, i)),
            pl.BlockSpec((Bn, Fin), lambda i: (i, 0)),
            pl.BlockSpec((Fin, H), lambda i: (0, 0)),
        ],
        out_specs=pl.BlockSpec((nch1, Bn, W), lambda i: (0, i, 0)),
        out_shape=jax.ShapeDtypeStruct((nch1, Np, W), jnp.float32),
    )(deg2, x_pad, W1)

    agg1 = _make_agg_kernel(Np, np1, nb_agg, rows_pt, 4)(g1, packed)

    def l2_body(deg_ref, a_ref, b_ref, w_ref, g_ref):
        i = pl.program_id(0)
        dinv = lax.rsqrt(deg_ref[0, :] + deg_ref[1, :] + 1.0)
        agg = jnp.concatenate([a_ref[k] for k in range(nch1)], axis=1)
        t = jnp.maximum(agg * dinv[:, None] + b_ref[...], 0.0)
        row = i * Bn + lax.broadcasted_iota(jnp.int32, (Bn, 1), 0)
        t = jnp.where(row < N, t, 0.0)
        g = jnp.dot(t, w_ref[...], preferred_element_type=jnp.float32)
        g = g * dinv[:, None]
        for k in range(nch2):
            g_ref[k] = g[:, W * k:W * (k + 1)]

    g2 = pl.pallas_call(
        l2_body,
        grid=(Np // Bn,),
        in_specs=[
            pl.BlockSpec((NC, Bn), lambda i: (0, i)),
            pl.BlockSpec((nch1, Bn, W), lambda i: (0, i, 0)),
            pl.BlockSpec((1, H), lambda i: (0, 0)),
            pl.BlockSpec((H, C), lambda i: (0, 0)),
        ],
        out_specs=pl.BlockSpec((nch2, Bn, W), lambda i: (0, i, 0)),
        out_shape=jax.ShapeDtypeStruct((nch2, Np, W), jnp.float32),
    )(deg2, agg1, b1r, W2)

    agg2 = _make_agg_kernel(Np, np2, nb_agg, rows_pt, 4)(g2, packed)

    def l3_body(deg_ref, a_ref, b_ref, o_ref):
        dinv = lax.rsqrt(deg_ref[0, :] + deg_ref[1, :] + 1.0)
        agg = jnp.concatenate([a_ref[k] for k in range(nch2)], axis=1)
        o_ref[...] = agg * dinv[:, None] + b_ref[...]

    out = pl.pallas_call(
        l3_body,
        grid=(Np // Bn,),
        in_specs=[
            pl.BlockSpec((NC, Bn), lambda i: (0, i)),
            pl.BlockSpec((nch2, Bn, W), lambda i: (0, i, 0)),
            pl.BlockSpec((1, C), lambda i: (0, 0)),
        ],
        out_specs=pl.BlockSpec((Bn, C), lambda i: (i, 0)),
        out_shape=jax.ShapeDtypeStruct((Np, C), jnp.float32),
    )(deg2, agg2, b2r)

    return out[:N]
